# Initial kernel scaffold; baseline (speedup 1.0000x reference)
import jax
import jax.numpy as jnp
from jax.experimental import pallas as pl


def kernel(user, item, theta_w, a_w, b_w, c_w):
    raise NotImplementedError("write your pallas kernel here")


# speedup vs baseline: 1.1817x; 1.1817x over previous
import functools

import jax
import jax.numpy as jnp
from jax import lax
from jax.experimental import pallas as pl
from jax.experimental.pallas import tpu as pltpu
from jax.experimental.pallas import tpu_sc as plsc

_BATCH = 16384
_L = 16

_LN2 = 0.6931471805599453


def _log_newton(t):
    bits = lax.bitcast_convert_type(t, jnp.int32)
    y = (bits.astype(jnp.float32) * (1.0 / (1 << 23)) - 126.94269504) * _LN2
    y = y - 1.0 + t * jnp.exp(-y)
    y = y - 1.0 + t * jnp.exp(-y)
    return y


def _irt_block(th, av, bv, cv):
    c = 1.0 / (1.0 + jnp.exp(-cv))
    t = 1.0 + jnp.exp(-jnp.abs(av))
    a = jnp.maximum(av, 0.0) + _log_newton(t)
    z = -1.73 * a * (th - bv) + 1e-08
    return c + (1.0 - c) / (1.0 + jnp.exp(z))


def _make_sc_kernel(num_cores, b_per_w):
    mesh = plsc.VectorSubcoreMesh(core_axis_name="c", subcore_axis_name="s")

    @functools.partial(
        pl.kernel,
        mesh=mesh,
        out_type=jax.ShapeDtypeStruct((_BATCH,), jnp.float32),
        scratch_types=[
            pltpu.VMEM((b_per_w,), jnp.int32),
            pltpu.VMEM((b_per_w,), jnp.int32),
            pltpu.VMEM((b_per_w,), jnp.float32),
            pltpu.VMEM((b_per_w,), jnp.float32),
            pltpu.VMEM((b_per_w,), jnp.float32),
            pltpu.VMEM((b_per_w,), jnp.float32),
            pltpu.VMEM((b_per_w,), jnp.float32),
            pltpu.SemaphoreType.DMA,
        ],
    )
    def k(user_hbm, item_hbm, theta_hbm, a_hbm, b_hbm, c_hbm, out_hbm,
          u_idx, i_idx, th_v, a_v, b_v, c_v, o_v, sem):
        wid = lax.axis_index("s") * num_cores + lax.axis_index("c")
        base = wid * b_per_w

        idx_cp_u = pltpu.make_async_copy(
            user_hbm.at[pl.ds(base, b_per_w)], u_idx, sem)
        idx_cp_i = pltpu.make_async_copy(
            item_hbm.at[pl.ds(base, b_per_w)], i_idx, sem)
        idx_cp_u.start()
        idx_cp_i.start()
        idx_cp_u.wait()
        idx_cp_i.wait()

        g_th = pltpu.make_async_copy(theta_hbm.at[u_idx], th_v, sem)
        g_a = pltpu.make_async_copy(a_hbm.at[i_idx], a_v, sem)
        g_b = pltpu.make_async_copy(b_hbm.at[i_idx], b_v, sem)
        g_c = pltpu.make_async_copy(c_hbm.at[i_idx], c_v, sem)
        g_th.start()
        g_a.start()
        g_b.start()
        g_c.start()
        g_th.wait()
        g_a.wait()
        g_b.wait()
        g_c.wait()

        for i in range(b_per_w // _L):
            s = pl.ds(i * _L, _L)
            o_v[s] = _irt_block(th_v[s], a_v[s], b_v[s], c_v[s])

        pltpu.sync_copy(o_v, out_hbm.at[pl.ds(base, b_per_w)])

    return k


def kernel(user, item, theta_w, a_w, b_w, c_w):
    info = plsc.get_sparse_core_info()
    num_workers = info.num_cores * info.num_subcores
    b_per_w = _BATCH // num_workers
    k = _make_sc_kernel(info.num_cores, b_per_w)
    return k(
        user.astype(jnp.int32),
        item.astype(jnp.int32),
        theta_w.reshape(-1),
        a_w.reshape(-1),
        b_w.reshape(-1),
        c_w.reshape(-1),
    )


# speedup vs baseline: 2.3058x; 1.9514x over previous
import functools

import jax
import jax.numpy as jnp
from jax import lax
from jax.experimental import pallas as pl
from jax.experimental.pallas import tpu as pltpu
from jax.experimental.pallas import tpu_sc as plsc

_BATCH = 16384
_L = 16

_LN2 = 0.6931471805599453


def _log_newton(t):
    bits = lax.bitcast_convert_type(t, jnp.int32)
    y = (bits.astype(jnp.float32) * (1.0 / (1 << 23)) - 126.94269504) * _LN2
    y = y - 1.0 + t * jnp.exp(-y)
    y = y - 1.0 + t * jnp.exp(-y)
    return y


def _irt_block(th, av, bv, cv):
    c = 1.0 / (1.0 + jnp.exp(-cv))
    t = 1.0 + jnp.exp(-jnp.abs(av))
    a = jnp.maximum(av, 0.0) + _log_newton(t)
    z = -1.73 * a * (th - bv) + 1e-08
    return c + (1.0 - c) / (1.0 + jnp.exp(z))


def _flatten_padded(w):
    n = w.shape[0]
    n_pad = -n % 1024
    if n_pad:
        w = jnp.pad(w, ((0, n_pad), (0, 0)))
    return w.reshape(-1)


def _make_sc_kernel(num_cores, b_per_w):
    mesh = plsc.VectorSubcoreMesh(core_axis_name="c", subcore_axis_name="s")

    @functools.partial(
        pl.kernel,
        mesh=mesh,
        out_type=jax.ShapeDtypeStruct((_BATCH,), jnp.float32),
        scratch_types=[
            pltpu.VMEM((b_per_w,), jnp.int32),
            pltpu.VMEM((b_per_w,), jnp.int32),
            pltpu.VMEM((b_per_w,), jnp.float32),
            pltpu.VMEM((b_per_w,), jnp.float32),
            pltpu.VMEM((b_per_w,), jnp.float32),
            pltpu.VMEM((b_per_w,), jnp.float32),
            pltpu.VMEM((b_per_w,), jnp.float32),
            pltpu.SemaphoreType.DMA,
        ],
    )
    def k(user_hbm, item_hbm, theta_hbm, a_hbm, b_hbm, c_hbm, out_hbm,
          u_idx, i_idx, th_v, a_v, b_v, c_v, o_v, sem):
        wid = lax.axis_index("s") * num_cores + lax.axis_index("c")
        base = wid * b_per_w

        idx_cp_u = pltpu.make_async_copy(
            user_hbm.at[pl.ds(base, b_per_w)], u_idx, sem)
        idx_cp_i = pltpu.make_async_copy(
            item_hbm.at[pl.ds(base, b_per_w)], i_idx, sem)
        idx_cp_u.start()
        idx_cp_i.start()
        idx_cp_u.wait()
        idx_cp_i.wait()

        g_th = pltpu.make_async_copy(theta_hbm.at[u_idx], th_v, sem)
        g_a = pltpu.make_async_copy(a_hbm.at[i_idx], a_v, sem)
        g_b = pltpu.make_async_copy(b_hbm.at[i_idx], b_v, sem)
        g_c = pltpu.make_async_copy(c_hbm.at[i_idx], c_v, sem)
        g_th.start()
        g_a.start()
        g_b.start()
        g_c.start()
        g_th.wait()
        g_a.wait()
        g_b.wait()
        g_c.wait()

        for i in range(b_per_w // _L):
            sl = pl.ds(i * _L, _L)
            o_v[sl] = _irt_block(th_v[sl], a_v[sl], b_v[sl], c_v[sl])

        pltpu.sync_copy(o_v, out_hbm.at[pl.ds(base, b_per_w)])

    return k


def kernel(user, item, theta_w, a_w, b_w, c_w):
    info = plsc.get_sparse_core_info()
    num_workers = info.num_cores * info.num_subcores
    b_per_w = _BATCH // num_workers
    k = _make_sc_kernel(info.num_cores, b_per_w)
    return k(
        user.astype(jnp.int32),
        item.astype(jnp.int32),
        _flatten_padded(theta_w),
        _flatten_padded(a_w),
        _flatten_padded(b_w),
        _flatten_padded(c_w),
    )
